# Initial kernel scaffold; baseline (speedup 1.0000x reference)
import jax
import jax.numpy as jnp
from jax.experimental import pallas as pl


def kernel(x, params, neigh_pix, neigh_weight):
    raise NotImplementedError("write your pallas kernel here")


# speedup vs baseline: 2.5813x; 2.5813x over previous
import jax
import jax.numpy as jnp
from jax import lax
from jax.experimental import pallas as pl
from jax.experimental.pallas import tpu as pltpu
from jax.experimental.pallas import tpu_sc as plsc

L = 10
KNB = 4
F = 16
C = 128
NC, NS = 2, 16
NW = NC * NS
R = L * KNB


def _make_sc_call(n_points):
    assert n_points % F == 0 and n_points >= C
    n_full = n_points // C
    tail = n_points % C
    n_chunks = n_full + (1 if tail else 0)
    iters = -(-n_chunks // NW)

    mesh = plsc.VectorSubcoreMesh(core_axis_name="c", subcore_axis_name="s")

    def body(idx_hbm, w_hbm, table_hbm, out_hbm,
             idx_v, w_v, rows_v, acc_v, sem_in, sem_g):
        wid = lax.axis_index("s") * NC + lax.axis_index("c")
        lanes = lax.iota(jnp.int32, F) * L

        def chunk_work(base, c_pts):
            cps = [pltpu.async_copy(idx_hbm.at[pl.ds(r * n_points + base, c_pts)],
                                    idx_v.at[r, pl.ds(0, c_pts)], sem_in)
                   for r in range(R)]
            cps += [pltpu.async_copy(w_hbm.at[pl.ds(r * n_points + base, c_pts)],
                                     w_v.at[r, pl.ds(0, c_pts)], sem_in)
                    for r in range(R)]
            for cp in cps:
                cp.wait()

            def level_body(l, carry2):
                gps = [
                    pltpu.async_copy(
                        table_hbm.at[idx_v.at[l * KNB + j, pl.ds(0, c_pts)]],
                        rows_v.at[pl.ds(j * C, c_pts)], sem_g)
                    for j in range(KNB)
                ]
                for gp in gps:
                    gp.wait()
                lane_l = lanes + l

                def grp_body(g, carry3):
                    n0 = g * F
                    wv = [w_v[l * KNB + j, pl.ds(n0, F)] for j in range(KNB)]
                    for i in range(F):
                        nn = n0 + i
                        acc = rows_v[0 * C + nn] * wv[0][i]
                        acc = acc + rows_v[1 * C + nn] * wv[1][i]
                        acc = acc + rows_v[2 * C + nn] * wv[2][i]
                        acc = acc + rows_v[3 * C + nn] * wv[3][i]
                        plsc.store_scatter(
                            acc_v, [jnp.full((F,), nn, jnp.int32), lane_l], acc)
                    return carry3

                lax.fori_loop(0, c_pts // F, grp_body, 0)
                return carry2

            lax.fori_loop(0, L, level_body, 0)
            pltpu.sync_copy(acc_v.at[pl.ds(0, c_pts)],
                            out_hbm.at[pl.ds(base, c_pts)])

        def chunk_body(t, carry):
            k = t * NW + wid
            @pl.when(k < n_full)
            def _full():
                chunk_work(k * C, C)

            if tail:
                @pl.when(k == n_full)
                def _tail():
                    chunk_work(n_full * C, tail)
            return carry

        lax.fori_loop(0, iters, chunk_body, 0)

    return pl.kernel(
        body,
        out_type=jax.ShapeDtypeStruct((n_points, F * L), jnp.float32),
        mesh=mesh,
        compiler_params=pltpu.CompilerParams(
            needs_layout_passes=False, use_tc_tiling_on_sc=False),
        scratch_types=[
            pltpu.VMEM((R, C), jnp.int32),
            pltpu.VMEM((R, C), jnp.float32),
            pltpu.VMEM((KNB * C, F), jnp.float32),
            pltpu.VMEM((C, F * L), jnp.float32),
            pltpu.SemaphoreType.DMA,
            pltpu.SemaphoreType.DMA,
        ],
    )


def kernel(x, params, neigh_pix, neigh_weight):
    n = x.shape[0]
    idx_flat = neigh_pix.astype(jnp.int32).reshape(-1)
    w_flat = neigh_weight.reshape(-1)
    run = _make_sc_call(n)
    return run(idx_flat, w_flat, params)


# speedup vs baseline: 2.5854x; 1.0016x over previous
import jax
import jax.numpy as jnp
from jax import lax
from jax.experimental import pallas as pl
from jax.experimental.pallas import tpu as pltpu
from jax.experimental.pallas import tpu_sc as plsc

L = 10
KNB = 4
F = 16
C = 256
G = 128
NC, NS = 2, 16
NW = NC * NS


def _splits(c_pts):
    segs = [(h * G, G) for h in range(c_pts // G)]
    if c_pts % G:
        segs.append((c_pts - c_pts % G, c_pts % G))
    return segs


def _make_sc_call(n_points):
    assert n_points % F == 0 and n_points >= C
    n_full = n_points // C
    tail = n_points % C
    n_chunks = n_full + (1 if tail else 0)
    iters = -(-n_chunks // NW)

    mesh = plsc.VectorSubcoreMesh(core_axis_name="c", subcore_axis_name="s")

    def body(idx_hbm, w_hbm, table_hbm, out_hbm,
             idx_v, w_v, rows_v, acc_v, sem_in, sem_g):
        wid = lax.axis_index("s") * NC + lax.axis_index("c")
        lanes = lax.iota(jnp.int32, F) * L

        def chunk_work(base, c_pts):
            cp_i = pltpu.async_copy(
                idx_hbm.at[:, :, pl.ds(base, c_pts)],
                idx_v.at[:, :, pl.ds(0, c_pts)], sem_in)
            cp_w = pltpu.async_copy(
                w_hbm.at[:, :, pl.ds(base, c_pts)],
                w_v.at[:, :, pl.ds(0, c_pts)], sem_in)
            cp_i.wait()
            cp_w.wait()

            def fire(l):
                buf = (l % 2) * KNB * C
                for j in range(KNB):
                    for off, sz in _splits(c_pts):
                        pltpu.async_copy(
                            table_hbm.at[idx_v.at[l, j, pl.ds(off, sz)]],
                            rows_v.at[pl.ds(buf + j * C + off, sz)], sem_g)

            def level_body(l, carry2):
                @pl.when(l < L)
                def _fire():
                    fire(l)

                @pl.when(l > 0)
                def _compute():
                    lp = l - 1
                    buf = (lp % 2) * KNB * C
                    pltpu.make_async_copy(
                        table_hbm.at[pl.ds(0, KNB * c_pts)],
                        rows_v.at[pl.ds(buf, KNB * c_pts)], sem_g).wait()
                    lane_l = lanes + lp

                    def grp_body(g, carry3):
                        n0 = g * F
                        wv = [w_v[lp, j, pl.ds(n0, F)] for j in range(KNB)]
                        for i in range(F):
                            nn = n0 + i
                            acc = rows_v[buf + 0 * C + nn] * wv[0][i]
                            acc = acc + rows_v[buf + 1 * C + nn] * wv[1][i]
                            acc = acc + rows_v[buf + 2 * C + nn] * wv[2][i]
                            acc = acc + rows_v[buf + 3 * C + nn] * wv[3][i]
                            plsc.store_scatter(
                                acc_v,
                                [jnp.full((F,), nn, jnp.int32), lane_l], acc)
                        return carry3

                    lax.fori_loop(0, c_pts // F, grp_body, 0)
                return carry2

            lax.fori_loop(0, L + 1, level_body, 0)
            pltpu.sync_copy(acc_v.at[pl.ds(0, c_pts)],
                            out_hbm.at[pl.ds(base, c_pts)])

        def chunk_body(t, carry):
            k = t * NW + wid

            @pl.when(k < n_full)
            def _full():
                chunk_work(k * C, C)

            if tail:
                @pl.when(k == n_full)
                def _tail():
                    chunk_work(n_full * C, tail)
            return carry

        lax.fori_loop(0, iters, chunk_body, 0)

    return pl.kernel(
        body,
        out_type=jax.ShapeDtypeStruct((n_points, F * L), jnp.float32),
        mesh=mesh,
        compiler_params=pltpu.CompilerParams(
            needs_layout_passes=False, use_tc_tiling_on_sc=False),
        scratch_types=[
            pltpu.VMEM((L, KNB, C), jnp.int32),
            pltpu.VMEM((L, KNB, C), jnp.float32),
            pltpu.VMEM((2 * KNB * C, F), jnp.float32),
            pltpu.VMEM((C, F * L), jnp.float32),
            pltpu.SemaphoreType.DMA,
            pltpu.SemaphoreType.DMA,
        ],
    )


def kernel(x, params, neigh_pix, neigh_weight):
    n = x.shape[0]
    run = _make_sc_call(n)
    return run(neigh_pix, neigh_weight, params)
